# baseline (speedup 1.0000x reference)
import jax
import jax.numpy as jnp
from jax.experimental import pallas as pl
from jax.experimental.pallas import tpu as pltpu

IN_DIM = 2048
OUT_DIM = 2048
NL = 8
N = 16384
BLK = 512
NB = N // BLK


def _celu(x):
    return jnp.where(x > 0, x, jnp.exp(jnp.minimum(x, 0.0)) - 1.0)


def _fused_kernel(idx_ref, x_ref, v_ref, t_ref, vm_ref,
                  W1_ref, b1_ref, gW2_ref, gb2_ref, W2bd_ref, b2_ref,
                  gW3_ref, gb3_ref, W3_ref, lb3_ref,
                  out_ref, eacc, sacc):
    i = pl.program_id(0)

    @pl.when(i == 0)
    def _init():
        eacc[...] = jnp.zeros_like(eacc)
        sacc[...] = jnp.zeros_like(sacc)
        out_ref[...] = jnp.zeros_like(out_ref)

    x = x_ref[...]
    v = v_ref[...]
    t = t_ref[...]
    idx = idx_ref[...]

    h1 = _celu(jnp.dot(x.astype(jnp.bfloat16), W1_ref[...],
                       preferred_element_type=jnp.float32)
               + b1_ref[...])
    hg1 = h1[:, :16]
    hl1 = h1[:, 16:]

    hg2 = _celu(jnp.dot(hg1, gW2_ref[...], preferred_element_type=jnp.float32)
                + gb2_ref[...])
    v_g = jnp.dot(hg2.astype(jnp.bfloat16), gW3_ref[...],
                  preferred_element_type=jnp.float32) \
        + gb3_ref[...]

    c16 = jax.lax.broadcasted_iota(jnp.int32, (BLK, NL * 16), 1) // 16
    hl1m = jnp.where(c16 == idx, hl1, 0.0)
    hl2 = _celu(jnp.dot(hl1m, W2bd_ref[...], preferred_element_type=jnp.float32)
                + b2_ref[...])
    c32 = jax.lax.broadcasted_iota(jnp.int32, (BLK, NL * 32), 1) // 32
    hl2m = jnp.where(c32 == idx, hl2, 0.0)

    oh = (jax.lax.broadcasted_iota(jnp.int32, (BLK, NL), 1)
          == idx).astype(jnp.float32)
    v_l = jnp.dot(hl2m.astype(jnp.bfloat16), W3_ref[...],
                  preferred_element_type=jnp.float32) \
        + jnp.dot(oh.astype(jnp.bfloat16), lb3_ref[...],
                  preferred_element_type=jnp.float32)

    d = jnp.sum(v_g * v_l, axis=1, keepdims=True)
    resid = v - v_g - v_l
    r = jnp.sum(resid * resid, axis=1, keepdims=True)
    ng = jnp.sqrt(jnp.sum(v_g * v_g, axis=1, keepdims=True))
    nl = jnp.sqrt(jnp.sum(v_l * v_l, axis=1, keepdims=True))
    tot = ng + nl
    s_g = ng / tot
    s_l = nl / tot

    vm = vm_ref[...]
    rn = jax.lax.rsqrt(jnp.sum(vm * vm, axis=1, keepdims=True))
    vmn = (vm * rn).astype(jnp.bfloat16)
    proj = jax.lax.dot_general(v_g.astype(jnp.bfloat16), vmn,
                               dimension_numbers=(((1,), (1,)), ((), ())),
                               preferred_element_type=jnp.float32)
    pm = jnp.mean(proj, axis=1, keepdims=True)
    pd = proj - pm
    stdt = jnp.sqrt(jnp.sum(pd * pd, axis=1, keepdims=True) / (NL - 1))

    cols = jnp.concatenate([d * d, r, jnp.ones_like(d)], axis=1)
    eacc[...] += jax.lax.dot_general(
        oh, cols, dimension_numbers=(((0,), (0,)), ((), ())),
        preferred_element_type=jnp.float32)

    parts = jnp.concatenate(
        [s_l, s_g, s_l * s_l, s_g * s_g, s_l * t, s_g * t, t, t * t, stdt],
        axis=1)
    sacc[...] += jnp.sum(parts, axis=0, keepdims=True)

    @pl.when(i == NB - 1)
    def _finalize():
        ea = eacc[...]
        sa = sacc[...]
        n = jnp.float32(N)
        cnt = ea[:, 2:3]
        loss_orth = jnp.sum(ea[:, 0:1] / cnt, keepdims=True)
        loss_recon = jnp.sum(ea[:, 1:2] / (cnt * OUT_DIM), keepdims=True)
        Sl, Sg = sa[0:1, 0:1], sa[0:1, 1:2]
        Sll, Sgg = sa[0:1, 2:3], sa[0:1, 3:4]
        Slt, Sgt = sa[0:1, 4:5], sa[0:1, 5:6]
        St, Stt, Sstd = sa[0:1, 6:7], sa[0:1, 7:8], sa[0:1, 8:9]
        var_t = Stt - St * St / n
        num_l = Slt - Sl * St / n
        den_l = jnp.sqrt(Sll - Sl * Sl / n) * jnp.sqrt(var_t) + 1e-8
        pcc_l = num_l / den_l
        num_g = Sgt - Sg * St / n
        den_g = jnp.sqrt(Sgg - Sg * Sg / n) * jnp.sqrt(var_t) + 1e-8
        pcc_g = num_g / den_g
        loss_pcc = -(jnp.where(pcc_l < 0.7, pcc_l, 0.0)
                     - jnp.where(pcc_g > -0.7, pcc_g, 0.0))
        loss_bal = Sstd / n
        out_ref[...] = jnp.concatenate(
            [loss_recon, loss_orth, loss_pcc, loss_bal], axis=1)


def kernel(v, x, idx, norm_t, v_mean, gW1, gb1, gW2, gb2, gW3, gb3,
           lW1, lb1, lW2, lb2, lW3, lb3):
    W1 = jnp.concatenate([gW1, lW1.reshape(NL * 16, IN_DIM)],
                         axis=0).T.astype(jnp.bfloat16)
    b1 = jnp.concatenate([gb1, lb1.reshape(NL * 16)]).reshape(1, -1)
    gW2T = gW2.T
    gb2r = gb2.reshape(1, -1)
    W2bd = jax.scipy.linalg.block_diag(*[lW2[e].T for e in range(NL)])
    b2 = lb2.reshape(1, NL * 32)
    gW3T = gW3.T.astype(jnp.bfloat16)
    gb3r = gb3.reshape(1, -1)
    W3 = jnp.transpose(lW3, (0, 2, 1)).reshape(NL * 32, OUT_DIM).astype(jnp.bfloat16)
    lb3h = lb3.astype(jnp.bfloat16)
    idx2 = idx.reshape(N, 1).astype(jnp.int32)

    row = lambda i: (i, 0)
    rep = lambda i: (0, 0)
    out = pl.pallas_call(
        _fused_kernel,
        grid=(NB,),
        in_specs=[
            pl.BlockSpec((BLK, 1), row),
            pl.BlockSpec((BLK, IN_DIM), row),
            pl.BlockSpec((BLK, OUT_DIM), row),
            pl.BlockSpec((BLK, 1), row),
            pl.BlockSpec((NL, OUT_DIM), rep),
            pl.BlockSpec((IN_DIM, 16 + NL * 16), rep),
            pl.BlockSpec((1, 16 + NL * 16), rep),
            pl.BlockSpec((16, 32), rep),
            pl.BlockSpec((1, 32), rep),
            pl.BlockSpec((NL * 16, NL * 32), rep),
            pl.BlockSpec((1, NL * 32), rep),
            pl.BlockSpec((32, OUT_DIM), rep),
            pl.BlockSpec((1, OUT_DIM), rep),
            pl.BlockSpec((NL * 32, OUT_DIM), rep),
            pl.BlockSpec((NL, OUT_DIM), rep),
        ],
        out_specs=pl.BlockSpec((1, 4), rep),
        out_shape=jax.ShapeDtypeStruct((1, 4), jnp.float32),
        scratch_shapes=[
            pltpu.VMEM((NL, 3), jnp.float32),
            pltpu.VMEM((1, 9), jnp.float32),
        ],
        compiler_params=pltpu.CompilerParams(
            dimension_semantics=("arbitrary",),
        ),
    )(idx2, x, v, norm_t, v_mean,
      W1, b1, gW2T, gb2r, W2bd, b2, gW3T, gb3r, W3, lb3h)
    return (out[0, 0], out[0, 1], out[0, 2], out[0, 3])


# speedup vs baseline: 1.2127x; 1.2127x over previous
import jax
import jax.numpy as jnp
from jax.experimental import pallas as pl
from jax.experimental.pallas import tpu as pltpu

IN_DIM = 2048
OUT_DIM = 2048
NL = 8
N = 16384
BLK = 512
NB = N // BLK
CD = 32 + NL * 32 + NL + 1


def _celu(x):
    return jnp.where(x > 0, x, jnp.exp(jnp.minimum(x, 0.0)) - 1.0)


def _dot(a, b):
    return jnp.dot(a, b, preferred_element_type=jnp.float32)


def _fused_kernel(idx_ref, x_ref, v_ref, t_ref, vm_ref,
                  W1_ref, b1_ref, gW2_ref, gb2_ref, W2bd_ref, b2_ref,
                  C_ref, Ct_ref,
                  out_ref, G_ref, pv_ref, eacc, sacc):
    i = pl.program_id(0)

    @pl.when(i == 0)
    def _init():
        eacc[...] = jnp.zeros_like(eacc)
        sacc[...] = jnp.zeros_like(sacc)
        out_ref[...] = jnp.zeros_like(out_ref)
        G_ref[...] = _dot(C_ref[...], Ct_ref[...]).astype(jnp.bfloat16)
        vm = vm_ref[...]
        rn = jax.lax.rsqrt(jnp.sum(vm * vm, axis=1, keepdims=True))
        vmn = (vm * rn).astype(jnp.bfloat16)
        pv_ref[...] = jax.lax.dot_general(
            C_ref[...], vmn, dimension_numbers=(((1,), (1,)), ((), ())),
            preferred_element_type=jnp.float32)

    x = x_ref[...]
    v = v_ref[...]
    t = t_ref[...]
    idx = idx_ref[...]

    h1 = _celu(_dot(x.astype(jnp.bfloat16), W1_ref[...]) + b1_ref[...])
    hg1 = h1[:, :16]
    hl1 = h1[:, 16:]

    hg2 = _celu(_dot(hg1, gW2_ref[...]) + gb2_ref[...])

    c16 = jax.lax.broadcasted_iota(jnp.int32, (BLK, NL * 16), 1) // 16
    hl1m = jnp.where(c16 == idx, hl1, 0.0)
    hl2 = _celu(_dot(hl1m, W2bd_ref[...]) + b2_ref[...])
    c32 = jax.lax.broadcasted_iota(jnp.int32, (BLK, NL * 32), 1) // 32
    hl2m = jnp.where(c32 == idx, hl2, 0.0)

    oh = (jax.lax.broadcasted_iota(jnp.int32, (BLK, NL), 1)
          == idx).astype(jnp.float32)

    us = jnp.concatenate([hg2, hl2m, oh, jnp.ones((BLK, 1), jnp.float32)],
                         axis=1)
    us_bf = us.astype(jnp.bfloat16)
    hg2_bf = us_bf[:, :32]
    Gb = G_ref[...]
    qg = _dot(hg2_bf, Gb[0:32, :]) \
        + Gb[CD - 1:CD, :].astype(jnp.float32)
    qs = _dot(us_bf, Gb)
    ql = qs - qg

    ng2 = jnp.sum(qg[:, 0:32] * hg2, axis=1, keepdims=True) \
        + qg[:, CD - 1:CD]
    d = jnp.sum(qg * us, axis=1, keepdims=True) - ng2
    nl2 = jnp.sum(ql * us, axis=1, keepdims=True) - d

    w = _dot(v.astype(jnp.bfloat16), Ct_ref[...])
    vs = jnp.sum(w * us, axis=1, keepdims=True)
    vv = jnp.sum(v * v, axis=1, keepdims=True)
    r = vv - 2.0 * vs + (ng2 + 2.0 * d + nl2)

    ng = jnp.sqrt(jnp.maximum(ng2, 0.0))
    nl = jnp.sqrt(jnp.maximum(nl2, 0.0))
    tot = ng + nl
    s_g = ng / tot
    s_l = nl / tot

    proj = _dot(hg2, pv_ref[0:32, :]) + pv_ref[CD - 1:CD, :]
    pm = jnp.mean(proj, axis=1, keepdims=True)
    pd = proj - pm
    stdt = jnp.sqrt(jnp.sum(pd * pd, axis=1, keepdims=True) / (NL - 1))

    cols = jnp.concatenate([d * d, r, jnp.ones_like(d)], axis=1)
    eacc[...] += jax.lax.dot_general(
        oh, cols, dimension_numbers=(((0,), (0,)), ((), ())),
        preferred_element_type=jnp.float32)

    parts = jnp.concatenate(
        [s_l, s_g, s_l * s_l, s_g * s_g, s_l * t, s_g * t, t, t * t, stdt],
        axis=1)
    sacc[...] += jnp.sum(parts, axis=0, keepdims=True)

    @pl.when(i == NB - 1)
    def _finalize():
        ea = eacc[...]
        sa = sacc[...]
        n = jnp.float32(N)
        cnt = ea[:, 2:3]
        loss_orth = jnp.sum(ea[:, 0:1] / cnt, keepdims=True)
        loss_recon = jnp.sum(ea[:, 1:2] / (cnt * OUT_DIM), keepdims=True)
        Sl, Sg = sa[0:1, 0:1], sa[0:1, 1:2]
        Sll, Sgg = sa[0:1, 2:3], sa[0:1, 3:4]
        Slt, Sgt = sa[0:1, 4:5], sa[0:1, 5:6]
        St, Stt, Sstd = sa[0:1, 6:7], sa[0:1, 7:8], sa[0:1, 8:9]
        var_t = Stt - St * St / n
        num_l = Slt - Sl * St / n
        den_l = jnp.sqrt(Sll - Sl * Sl / n) * jnp.sqrt(var_t) + 1e-8
        pcc_l = num_l / den_l
        num_g = Sgt - Sg * St / n
        den_g = jnp.sqrt(Sgg - Sg * Sg / n) * jnp.sqrt(var_t) + 1e-8
        pcc_g = num_g / den_g
        loss_pcc = -(jnp.where(pcc_l < 0.7, pcc_l, 0.0)
                     - jnp.where(pcc_g > -0.7, pcc_g, 0.0))
        loss_bal = Sstd / n
        out_ref[...] = jnp.concatenate(
            [loss_recon, loss_orth, loss_pcc, loss_bal], axis=1)


def kernel(v, x, idx, norm_t, v_mean, gW1, gb1, gW2, gb2, gW3, gb3,
           lW1, lb1, lW2, lb2, lW3, lb3):
    W1 = jnp.concatenate([gW1, lW1.reshape(NL * 16, IN_DIM)],
                         axis=0).T.astype(jnp.bfloat16)
    b1 = jnp.concatenate([gb1, lb1.reshape(NL * 16)]).reshape(1, -1)
    gW2T = gW2.T
    gb2r = gb2.reshape(1, -1)
    W2bd = jax.scipy.linalg.block_diag(*[lW2[e].T for e in range(NL)])
    b2 = lb2.reshape(1, NL * 32)
    W3 = jnp.transpose(lW3, (0, 2, 1)).reshape(NL * 32, OUT_DIM)
    C = jnp.concatenate([gW3.T, W3, lb3, gb3.reshape(1, OUT_DIM)], axis=0)
    C_bf = C.astype(jnp.bfloat16)
    Ct_bf = C.T.astype(jnp.bfloat16)
    idx2 = idx.reshape(N, 1).astype(jnp.int32)

    row = lambda i: (i, 0)
    rep = lambda i: (0, 0)
    out = pl.pallas_call(
        _fused_kernel,
        grid=(NB,),
        in_specs=[
            pl.BlockSpec((BLK, 1), row),
            pl.BlockSpec((BLK, IN_DIM), row),
            pl.BlockSpec((BLK, OUT_DIM), row),
            pl.BlockSpec((BLK, 1), row),
            pl.BlockSpec((NL, OUT_DIM), rep),
            pl.BlockSpec((IN_DIM, 16 + NL * 16), rep),
            pl.BlockSpec((1, 16 + NL * 16), rep),
            pl.BlockSpec((16, 32), rep),
            pl.BlockSpec((1, 32), rep),
            pl.BlockSpec((NL * 16, NL * 32), rep),
            pl.BlockSpec((1, NL * 32), rep),
            pl.BlockSpec((CD, OUT_DIM), rep),
            pl.BlockSpec((OUT_DIM, CD), rep),
        ],
        out_specs=pl.BlockSpec((1, 4), rep),
        out_shape=jax.ShapeDtypeStruct((1, 4), jnp.float32),
        scratch_shapes=[
            pltpu.VMEM((CD, CD), jnp.bfloat16),
            pltpu.VMEM((CD, NL), jnp.float32),
            pltpu.VMEM((NL, 3), jnp.float32),
            pltpu.VMEM((1, 9), jnp.float32),
        ],
        compiler_params=pltpu.CompilerParams(
            dimension_semantics=("arbitrary",),
        ),
    )(idx2, x, v, norm_t, v_mean,
      W1, b1, gW2T, gb2r, W2bd, b2, C_bf, Ct_bf)
    return (out[0, 0], out[0, 1], out[0, 2], out[0, 3])
